# baseline (speedup 1.0000x reference)
import functools
import math

import jax
import jax.numpy as jnp
from jax import lax
from jax.experimental import pallas as pl
from jax.experimental.pallas import tpu as pltpu
from jax.experimental.pallas import tpu_sc as plsc

D_MODEL = 64
SCALE = math.sqrt(D_MODEL)
NC = 2
NS = 16
NW = NC * NS
SEQ = 200
BATCH = 4096
PB = 128
NBLK = BATCH // PB
NPANEL = SEQ * NBLK
WP = NPANEL // NW

_mesh = plsc.VectorSubcoreMesh(core_axis_name="c", subcore_axis_name="s")


@functools.partial(
    pl.kernel,
    out_type=jax.ShapeDtypeStruct((SEQ, D_MODEL, BATCH), jnp.float32),
    mesh=_mesh,
    scratch_types=[
        pltpu.VMEM((2, PB), jnp.int32),
        pltpu.VMEM((2, PB, D_MODEL), jnp.float32),
        pltpu.VMEM((2, D_MODEL, PB), jnp.float32),
        pltpu.SemaphoreType.DMA,
        pltpu.SemaphoreType.DMA,
    ],
    compiler_params=pltpu.CompilerParams(
        use_tc_tiling_on_sc=False, needs_layout_passes=False
    ),
)
def _emb_lookup(xt_hbm, lut_hbm, out_hbm, idx_v, rows_v, tr_v, gsem, wsem):
    wid = lax.axis_index("s") * NC + lax.axis_index("c")
    pbase = wid * WP
    iota16 = lax.iota(jnp.int32, 16)

    def fire(p, buf):
        s = p // NBLK
        blk = p % NBLK
        pltpu.sync_copy(xt_hbm.at[s, pl.ds(blk * PB, PB)], idx_v.at[buf])
        pltpu.async_copy(lut_hbm.at[idx_v.at[buf]], rows_v.at[buf], gsem)

    fire(pbase, 0)

    @pl.loop(0, WP, step=2)
    def _(pp):
        for b in range(2):
            lp = pp + b
            p = pbase + lp
            rows_b = rows_v.at[b]
            tr_b = tr_v.at[b]

            @pl.when(lp > 0)
            def _():
                pltpu.make_async_copy(
                    tr_v.at[1 - b], out_hbm.at[0, :, pl.ds(0, PB)], wsem
                ).wait()

            @pl.when(lp + 1 < WP)
            def _():
                fire(p + 1, 1 - b)

            pltpu.make_async_copy(
                lut_hbm.at[pl.ds(0, PB)], rows_b, gsem
            ).wait()

            @plsc.parallel_loop(0, PB, unroll=2)
            def _(r):
                rcol = jnp.full((16,), r, dtype=jnp.int32)
                for q in range(D_MODEL // 16):
                    v = rows_b[r, pl.ds(q * 16, 16)] * SCALE
                    plsc.store_scatter(tr_b, [iota16 + q * 16, rcol], v)

            s = p // NBLK
            blk = p % NBLK
            pltpu.async_copy(
                tr_b, out_hbm.at[s, :, pl.ds(blk * PB, PB)], wsem
            )

    pltpu.make_async_copy(
        tr_v.at[1], out_hbm.at[0, :, pl.ds(0, PB)], wsem
    ).wait()


def kernel(x, lut):
    xt = x.astype(jnp.int32).T
    out_phys = _emb_lookup(xt, lut)
    return jnp.transpose(out_phys, (2, 0, 1))


# speedup vs baseline: 1.2111x; 1.2111x over previous
import functools
import math

import jax
import jax.numpy as jnp
from jax import lax
from jax.experimental import pallas as pl
from jax.experimental.pallas import tpu as pltpu
from jax.experimental.pallas import tpu_sc as plsc

D_MODEL = 64
SCALE = math.sqrt(D_MODEL)
NC = 2
NS = 16
NW = NC * NS
SEQ = 200
BATCH = 4096
VOCAB2 = 500000
PB = 128
NBLK = BATCH // PB
NPANEL = SEQ * NBLK
WP = NPANEL // NW

_mesh = plsc.VectorSubcoreMesh(core_axis_name="c", subcore_axis_name="s")


@functools.partial(
    pl.kernel,
    out_type=jax.ShapeDtypeStruct((SEQ, D_MODEL, BATCH), jnp.float32),
    mesh=_mesh,
    scratch_types=[
        pltpu.VMEM((2, PB), jnp.int32),
        pltpu.VMEM((2, PB), jnp.int32),
        pltpu.VMEM((2, PB), jnp.int32),
        pltpu.VMEM((2, PB, 128), jnp.float32),
        pltpu.VMEM((2, D_MODEL, PB), jnp.float32),
        pltpu.SemaphoreType.DMA,
        pltpu.SemaphoreType.DMA,
    ],
    compiler_params=pltpu.CompilerParams(
        use_tc_tiling_on_sc=True, needs_layout_passes=False
    ),
)
def _emb_lookup(xt_hbm, lut_hbm, out_hbm, idx_v, row_v, half_v, rows_v, tr_v,
                gsem, wsem):
    wid = lax.axis_index("s") * NC + lax.axis_index("c")
    pbase = wid * WP
    iota16 = lax.iota(jnp.int32, 16)

    def fire(p, buf):
        s = p // NBLK
        blk = p % NBLK
        pltpu.sync_copy(xt_hbm.at[s, pl.ds(blk * PB, PB)], idx_v.at[buf])
        for k in range(PB // 16):
            iv = idx_v[buf, pl.ds(k * 16, 16)]
            row_v[buf, pl.ds(k * 16, 16)] = lax.shift_right_logical(iv, 1)
            half_v[buf, pl.ds(k * 16, 16)] = lax.shift_left(
                jnp.bitwise_and(iv, 1), 6
            )
        pltpu.async_copy(lut_hbm.at[row_v.at[buf]], rows_v.at[buf], gsem)

    fire(pbase, 0)

    @pl.loop(0, WP, step=2)
    def _(pp):
        for b in range(2):
            lp = pp + b
            p = pbase + lp
            rows_b = rows_v.at[b]
            tr_b = tr_v.at[b]

            @pl.when(lp > 0)
            def _():
                pltpu.make_async_copy(
                    tr_v.at[1 - b], out_hbm.at[0, :, pl.ds(0, PB)], wsem
                ).wait()

            @pl.when(lp + 1 < WP)
            def _():
                fire(p + 1, 1 - b)

            pltpu.make_async_copy(
                lut_hbm.at[pl.ds(0, PB)], rows_b, gsem
            ).wait()

            @pl.loop(0, PB // 16)
            def _(r0):
                rvec = r0 * 16 + iota16
                hv = half_v[b, pl.ds(r0 * 16, 16)]
                for d in range(16):
                    diag = jnp.bitwise_and(iota16 + d, 15)
                    for f0 in range(0, D_MODEL, 16):
                        jvec = diag + f0
                        v = plsc.load_gather(rows_b, [rvec, hv + jvec])
                        plsc.store_scatter(tr_b, [jvec, rvec], v * SCALE)

            s = p // NBLK
            blk = p % NBLK
            pltpu.async_copy(
                tr_b, out_hbm.at[s, :, pl.ds(blk * PB, PB)], wsem
            )

    pltpu.make_async_copy(
        tr_v.at[1], out_hbm.at[0, :, pl.ds(0, PB)], wsem
    ).wait()


def kernel(x, lut):
    xt = x.astype(jnp.int32).T
    lut2 = lut.reshape(VOCAB2, 128)
    out_phys = _emb_lookup(xt, lut2)
    return jnp.transpose(out_phys, (2, 0, 1))


# speedup vs baseline: 1.5370x; 1.2691x over previous
import functools
import math

import jax
import jax.numpy as jnp
from jax import lax
from jax.experimental import pallas as pl
from jax.experimental.pallas import tpu as pltpu
from jax.experimental.pallas import tpu_sc as plsc

D_MODEL = 64
SCALE = math.sqrt(D_MODEL)
NC = 2
NS = 16
NW = NC * NS
SEQ = 200
BATCH = 4096
VOCAB2 = 500000
PB = 128
NBLK = BATCH // PB
NPANEL = SEQ * NBLK
WP = NPANEL // NW

_mesh = plsc.VectorSubcoreMesh(core_axis_name="c", subcore_axis_name="s")


@functools.partial(
    pl.kernel,
    out_type=jax.ShapeDtypeStruct((SEQ, D_MODEL, BATCH), jnp.float32),
    mesh=_mesh,
    scratch_types=[
        pltpu.VMEM((2, PB), jnp.int32),
        pltpu.VMEM((2, PB), jnp.int32),
        pltpu.VMEM((2, PB), jnp.int32),
        pltpu.VMEM((2, PB, 128), jnp.float32),
        pltpu.VMEM((2, D_MODEL, PB), jnp.float32),
        pltpu.SemaphoreType.DMA,
        pltpu.SemaphoreType.DMA,
    ],
    compiler_params=pltpu.CompilerParams(
        use_tc_tiling_on_sc=True,
        needs_layout_passes=False,
        disable_bounds_checks=True,
    ),
)
def _emb_lookup(xt_hbm, lut_hbm, out_hbm, idx_v, row_v, half_v, rows_v, tr_v,
                gsem, wsem):
    wid = lax.axis_index("s") * NC + lax.axis_index("c")
    pbase = wid * WP
    iota16 = lax.iota(jnp.int32, 16)

    def fire(p, buf):
        s = p // NBLK
        blk = p % NBLK
        pltpu.sync_copy(xt_hbm.at[s, pl.ds(blk * PB, PB)], idx_v.at[buf])
        for k in range(PB // 16):
            iv = idx_v[buf, pl.ds(k * 16, 16)]
            row_v[buf, pl.ds(k * 16, 16)] = lax.shift_right_logical(iv, 1)
            half_v[buf, pl.ds(k * 16, 16)] = lax.shift_left(
                jnp.bitwise_and(iv, 1), 6
            )
        pltpu.async_copy(lut_hbm.at[row_v.at[buf]], rows_v.at[buf], gsem)

    fire(pbase, 0)

    @pl.loop(0, WP, step=2)
    def _(pp):
        for b in range(2):
            lp = pp + b
            p = pbase + lp
            rows_b = rows_v.at[b]
            tr_b = tr_v.at[b]

            @pl.when(lp > 0)
            def _():
                pltpu.make_async_copy(
                    tr_v.at[1 - b], out_hbm.at[0, :, pl.ds(0, PB)], wsem
                ).wait()

            @pl.when(lp + 1 < WP)
            def _():
                fire(p + 1, 1 - b)

            pltpu.make_async_copy(
                lut_hbm.at[pl.ds(0, PB)], rows_b, gsem
            ).wait()

            @plsc.parallel_loop(0, PB // 16, unroll=2)
            def _(r0):
                rvec = r0 * 16 + iota16
                hv = half_v[b, pl.ds(r0 * 16, 16)]
                for d in range(16):
                    diag = jnp.bitwise_and(iota16 + d, 15)
                    for f0 in range(0, D_MODEL, 16):
                        jvec = diag + f0
                        v = plsc.load_gather(rows_b, [rvec, hv + jvec])
                        plsc.store_scatter(tr_b, [jvec, rvec], v * SCALE)

            s = p // NBLK
            blk = p % NBLK
            pltpu.async_copy(
                tr_b, out_hbm.at[s, :, pl.ds(blk * PB, PB)], wsem
            )

    pltpu.make_async_copy(
        tr_v.at[1], out_hbm.at[0, :, pl.ds(0, PB)], wsem
    ).wait()


def kernel(x, lut):
    xt = x.astype(jnp.int32).T
    lut2 = lut.reshape(VOCAB2, 128)
    out_phys = _emb_lookup(xt, lut2)
    return jnp.transpose(out_phys, (2, 0, 1))
